# baseline (device time: 99300 ns/iter reference)
import jax
import jax.numpy as jnp
from jax import lax
from jax.experimental import pallas as pl
from jax.experimental.pallas import tpu as pltpu

N_DEV = 8
SUBS = 4

OFFS = (7, 1, 6, 2, 5, 3, 4, 0)

BOUND = 8.0
S = 32760.0 / BOUND


def kernel(x, w_mat):
    m, k = x.shape
    _, n = w_mat.shape
    m_per = m // N_DEV
    nq = n // 4
    rw = m_per // SUBS

    def body(x_ref, w_ref, out_ref, xb_ref, wb_ref, xstage_ref,
             outv_ref, commr_ref, comml_ref, amax_ref,
             x_sems, out_sems,
             sendr_sems, recvr_sems, sendl_sems, recvl_sems,
             ax_send_sems, ax_recv_sems):
        my = lax.axis_index("i")
        left = (my - 1 + N_DEV) % N_DEV
        right = (my + 1) % N_DEV

        def w_copy(qt):
            cp = pltpu.make_async_copy(
                w_ref.at[:, pl.ds(qt * nq, nq)],
                xstage_ref.at[qt % 2], x_sems.at[qt % 2])
            cp.start()
            return cp

        def x_copy(t):
            c = (my + OFFS[t]) % N_DEV
            cp = pltpu.make_async_copy(
                x_ref.at[pl.ds(c * m_per, m_per), :],
                xstage_ref.at[t % 2], x_sems.at[t % 2])
            cp.start()
            return cp

        wcp = [None] * 4
        wcp[0] = w_copy(0)
        wcp[1] = w_copy(1)
        xcp = [None] * N_DEV

        def x_cast(t):
            xcp[t].wait()
            c = (my + OFFS[t]) % N_DEV
            xb_ref[pl.ds(c * m_per, m_per), :] = (
                xstage_ref[t % 2].astype(jnp.bfloat16))
            if t + 2 < N_DEV:
                xcp[t + 2] = x_copy(t + 2)

        barrier_sem = pltpu.get_barrier_semaphore()
        for nbr in (left, right):
            pl.semaphore_signal(
                barrier_sem, inc=1,
                device_id=(nbr,), device_id_type=pl.DeviceIdType.MESH,
            )
        pl.semaphore_wait(barrier_sem, 2)

        for qt in range(4):
            wcp[qt].wait()
            wb_ref[:, qt * nq:(qt + 1) * nq] = (
                xstage_ref[qt % 2].astype(jnp.bfloat16))
            if qt + 2 < 4:
                wcp[qt + 2] = w_copy(qt + 2)
            else:
                xcp[qt - 2] = x_copy(qt - 2)
        x_cast(0)
        x_cast(1)

        def p_qs(idx, q, s):
            return jnp.dot(xb_ref[pl.ds(idx * m_per + s * rw, rw), :],
                           wb_ref[:, pl.ds(q * nq, nq)],
                           preferred_element_type=jnp.float32)

        def enc(v):
            u = jnp.clip(v * S, -32760.0, 32760.0) + 32768.5
            return u.astype(jnp.int32)

        def dec(c):
            return (c.astype(jnp.float32) - 32768.0) * (1.0 / S)

        def pack(vlo, vhi):
            return jnp.left_shift(enc(vhi), 16) | enc(vlo)

        def unpack(p):
            vlo = dec(p & 0xFFFF)
            vhi = dec(jnp.right_shift(p, 16) & 0xFFFF)
            return vlo, vhi

        def mk_rdma(cref, ssems, rsems, h, s, dev):
            src_slot = 7 if h == 0 else h - 1
            return pltpu.make_async_remote_copy(
                src_ref=cref.at[src_slot, pl.ds(s * rw, rw), :],
                dst_ref=cref.at[h, pl.ds(s * rw, rw), :],
                send_sem=ssems.at[h, s],
                recv_sem=rsems.at[h, s],
                device_id=(dev,),
                device_id_type=pl.DeviceIdType.MESH,
            )

        jr = (my - 1 + N_DEV) % N_DEV
        jl = (my + 1) % N_DEV
        all_sends = []
        cur_r = [None] * SUBS
        cur_l = [None] * SUBS
        for s in range(SUBS):
            rs = slice(s * rw, (s + 1) * rw)
            commr_ref[7, rs, :] = pack(p_qs(jr, 0, s), p_qs(jr, 1, s))
            rr = mk_rdma(commr_ref, sendr_sems, recvr_sems, 0, s, right)
            rr.start()
            comml_ref[7, rs, :] = pack(p_qs(jl, 2, s), p_qs(jl, 3, s))
            rl = mk_rdma(comml_ref, sendl_sems, recvl_sems, 0, s, left)
            rl.start()
            cur_r[s], cur_l[s] = rr, rl
            all_sends += [rr, rl]

        ys = {}
        for h in range(N_DEV - 1):
            if 2 + 2 * h < N_DEV:
                x_cast(2 + 2 * h)
                x_cast(3 + 2 * h)
            cr = (my - 2 - h + 2 * N_DEV) % N_DEV
            cl = (my + 2 + h) % N_DEV
            nxt_r = [None] * SUBS
            nxt_l = [None] * SUBS
            for s in range(SUBS):
                rs = slice(s * rw, (s + 1) * rw)
                a0, a1 = p_qs(cr, 0, s), p_qs(cr, 1, s)
                a2, a3 = p_qs(cl, 2, s), p_qs(cl, 3, s)
                cur_r[s].wait_recv()
                cur_l[s].wait_recv()
                v0, v1 = unpack(commr_ref[h, rs, :])
                v2, v3 = unpack(comml_ref[h, rs, :])
                if h < N_DEV - 2:
                    commr_ref[h, rs, :] = pack(v0 + a0, v1 + a1)
                    comml_ref[h, rs, :] = pack(v2 + a2, v3 + a3)
                    rr = mk_rdma(commr_ref, sendr_sems, recvr_sems,
                                 h + 1, s, right)
                    rl = mk_rdma(comml_ref, sendl_sems, recvl_sems,
                                 h + 1, s, left)
                    rr.start()
                    rl.start()
                    nxt_r[s], nxt_l[s] = rr, rl
                    all_sends += [rr, rl]
                else:
                    ys[(0, s)] = v0 + a0
                    ys[(1, s)] = v1 + a1
                    ys[(2, s)] = v2 + a2
                    ys[(3, s)] = v3 + a3
            cur_r, cur_l = nxt_r, nxt_l

        local_amax = jnp.max(jnp.stack(
            [jnp.max(jnp.abs(v)) for v in ys.values()]))
        amax_ref[pl.ds(my, 1)] = jnp.full((1, 8, 128), local_amax,
                                          dtype=jnp.float32)
        ax_rdmas = []
        for off in range(1, N_DEV):
            tgt = (my + off) % N_DEV
            r = pltpu.make_async_remote_copy(
                src_ref=amax_ref.at[my],
                dst_ref=amax_ref.at[my],
                send_sem=ax_send_sems.at[off],
                recv_sem=ax_recv_sems.at[my],
                device_id=(tgt,),
                device_id_type=pl.DeviceIdType.MESH,
            )
            r.start()
            ax_rdmas.append(r)
        for off in range(1, N_DEV):
            src = (my + off) % N_DEV
            pltpu.make_async_remote_copy(
                src_ref=amax_ref.at[src],
                dst_ref=amax_ref.at[src],
                send_sem=ax_send_sems.at[off],
                recv_sem=ax_recv_sems.at[src],
                device_id=(my,),
                device_id_type=pl.DeviceIdType.MESH,
            ).wait_recv()
        gmax = jnp.max(amax_ref[:, :, :])

        scale = gmax / 448.0
        inv_scale = 448.0 / gmax
        ocp = []
        for s in range(SUBS):
            rs = slice(s * rw, (s + 1) * rw)
            for q in range(4):
                c = jnp.clip(ys[(q, s)] * inv_scale, -448.0, 448.0
                             ).astype(jnp.float8_e4m3fn)
                outv_ref[rs, q * nq:(q + 1) * nq] = (
                    c.astype(jnp.float32) * scale).astype(jnp.bfloat16)
            cp = pltpu.make_async_copy(
                outv_ref.at[pl.ds(s * rw, rw), :],
                out_ref.at[pl.ds(s * rw, rw), :], out_sems.at[s])
            cp.start()
            ocp.append(cp)
        for cp in ocp:
            cp.wait()

        for r in ax_rdmas:
            r.wait_send()
        for r in all_sends:
            r.wait_send()

    return pl.pallas_call(
        body,
        out_shape=jax.ShapeDtypeStruct((m_per, n), jnp.bfloat16),
        in_specs=[
            pl.BlockSpec(memory_space=pl.ANY),
            pl.BlockSpec(memory_space=pl.ANY),
        ],
        out_specs=pl.BlockSpec(memory_space=pl.ANY),
        scratch_shapes=[
            pltpu.VMEM((m, k), jnp.bfloat16),
            pltpu.VMEM((k, n), jnp.bfloat16),
            pltpu.VMEM((2, m_per, k), jnp.float32),
            pltpu.VMEM((m_per, n), jnp.bfloat16),
            pltpu.VMEM((N_DEV, m_per, nq), jnp.int32),
            pltpu.VMEM((N_DEV, m_per, nq), jnp.int32),
            pltpu.VMEM((N_DEV, 8, 128), jnp.float32),
            pltpu.SemaphoreType.DMA((2,)),
            pltpu.SemaphoreType.DMA((SUBS,)),
            pltpu.SemaphoreType.DMA((N_DEV - 1, SUBS)),
            pltpu.SemaphoreType.DMA((N_DEV - 1, SUBS)),
            pltpu.SemaphoreType.DMA((N_DEV - 1, SUBS)),
            pltpu.SemaphoreType.DMA((N_DEV - 1, SUBS)),
            pltpu.SemaphoreType.DMA((N_DEV,)),
            pltpu.SemaphoreType.DMA((N_DEV,)),
        ],
        compiler_params=pltpu.CompilerParams(collective_id=0),
    )(x, w_mat)


# device time: 92112 ns/iter; 1.0780x vs baseline; 1.0780x over previous
import jax
import jax.numpy as jnp
from jax import lax
from jax.experimental import pallas as pl
from jax.experimental.pallas import tpu as pltpu

N_DEV = 8

GROWS = ((0, 176), (176, 344), (344, 512))
GORDER = ((0, 1, 2), (1, 2, 0), (2, 0, 1))
MASKD = (1, 3, 4)

E_ORDER = (1, 2, 3, 4, 5, 6, 7, 0)


def _D(d, e):
    if d == 0:
        return (e ^ (e >> 1)) & 1
    if d == 1:
        return (e >> 1) & 1
    return (e >> 2) & 1


KEPT4 = tuple(tuple(e for e in range(8) if _D(GORDER[g][0], e) == 0)
              for g in range(3))
KEPT2 = tuple(tuple(e for e in KEPT4[g] if _D(GORDER[g][1], e) == 0)
              for g in range(3))
SENT2 = tuple(tuple(e for e in KEPT4[g] if _D(GORDER[g][1], e) == 1)
              for g in range(3))
SENT1 = tuple(tuple(e for e in KEPT2[g] if _D(GORDER[g][2], e) == 1)
              for g in range(3))

BOUND = 8.0
S = 32760.0 / BOUND


def kernel(x, w_mat):
    m, k = x.shape
    _, n = w_mat.shape
    m_per = m // N_DEV
    n2 = n // 2

    def body(x_ref, w_ref, out_ref, xb_ref, wb_ref, stage_ref, sendbuf_ref,
             work_ref, r2_ref, r3_ref, amax_ref,
             stage_sems, s1s_sems, s1r_sems, s2s_sems, s2r_sems,
             s3s_sems, s3r_sems, ax_send_sems, ax_recv_sems):
        my = lax.axis_index("i")

        def w_copy(qt):
            cp = pltpu.make_async_copy(
                w_ref.at[:, pl.ds(qt * 512, 512)],
                stage_ref.at[qt % 2], stage_sems.at[qt % 2])
            cp.start()
            return cp

        def x_copy(t):
            c = my ^ E_ORDER[t]
            cp = pltpu.make_async_copy(
                x_ref.at[pl.ds(c * m_per, m_per), :],
                stage_ref.at[t % 2], stage_sems.at[t % 2])
            cp.start()
            return cp

        wcp = [None] * 4
        wcp[0] = w_copy(0)
        wcp[1] = w_copy(1)
        xcp = [None] * N_DEV

        def x_cast(t):
            xcp[t].wait()
            c = my ^ E_ORDER[t]
            xb_ref[pl.ds(c * m_per, m_per), :] = (
                stage_ref[t % 2].astype(jnp.bfloat16))
            if t + 2 < N_DEV:
                xcp[t + 2] = x_copy(t + 2)

        barrier_sem = pltpu.get_barrier_semaphore()
        for mk in MASKD:
            pl.semaphore_signal(
                barrier_sem, inc=1,
                device_id=(my ^ mk,), device_id_type=pl.DeviceIdType.MESH,
            )

        for qt in range(4):
            wcp[qt].wait()
            wb_ref[:, qt * 512:(qt + 1) * 512] = (
                stage_ref[qt % 2].astype(jnp.bfloat16))
            if qt + 2 < 4:
                wcp[qt + 2] = w_copy(qt + 2)
            else:
                xcp[qt - 2] = x_copy(qt - 2)

        pl.semaphore_wait(barrier_sem, 3)

        def grows_slice(g):
            g0, g1 = GROWS[g]
            return pl.ds(g0, g1 - g0)

        def piece(e, g):
            g0, g1 = GROWS[g]
            j = my ^ e
            return jnp.dot(
                xb_ref[pl.ds(j * m_per + g0, g1 - g0), :], wb_ref[:, :],
                preferred_element_type=jnp.float32)

        def enc(v):
            u = jnp.clip(v * S, -32760.0, 32760.0) + 32768.5
            return u.astype(jnp.int32) - 32768

        def pack(v):
            return (jnp.left_shift(enc(v[:, n2:]), 16)
                    | (enc(v[:, :n2]) & 0xFFFF))

        def lanes(pp):
            lo = jnp.right_shift(jnp.left_shift(pp, 16), 16)
            hi = jnp.right_shift(pp, 16)
            return lo, hi

        def iadd(a, b):
            alo, ahi = lanes(a)
            blo, bhi = lanes(b)
            return (jnp.left_shift(ahi + bhi, 16) | ((alo + blo) & 0xFFFF))

        all_sends = []

        recv_s1 = {}
        for g in range(3):
            for kk in range(4):
                recv_s1[(g, kk)] = pltpu.make_async_remote_copy(
                    src_ref=work_ref.at[kk, grows_slice(g), :],
                    dst_ref=work_ref.at[kk, grows_slice(g), :],
                    send_sem=s1s_sems.at[g, kk],
                    recv_sem=s1r_sems.at[g, kk],
                    device_id=(my ^ MASKD[GORDER[g][0]],),
                    device_id_type=pl.DeviceIdType.MESH,
                )
        sb_inflight = [None] * 4
        sb_i = 0
        scount = [0, 0, 0]
        for t in range(N_DEV):
            x_cast(t)
            e = E_ORDER[t]
            if e == 0:
                continue
            for g in range(3):
                d0 = GORDER[g][0]
                if _D(d0, e) != 1:
                    continue
                mk = MASKD[d0]
                k_dst = KEPT4[g].index(e ^ mk)
                g0, g1 = GROWS[g]
                rows = g1 - g0
                if sb_inflight[sb_i] is not None:
                    sb_inflight[sb_i].wait_send()
                sendbuf_ref[sb_i, 0:rows, :] = pack(piece(e, g))
                rr = pltpu.make_async_remote_copy(
                    src_ref=sendbuf_ref.at[sb_i, pl.ds(0, rows), :],
                    dst_ref=work_ref.at[k_dst, grows_slice(g), :],
                    send_sem=s1s_sems.at[g, scount[g]],
                    recv_sem=s1r_sems.at[g, k_dst],
                    device_id=(my ^ mk,),
                    device_id_type=pl.DeviceIdType.MESH,
                )
                rr.start()
                sb_inflight[sb_i] = rr
                sb_i = (sb_i + 1) % 4
                scount[g] += 1
        for kk in range(4):
            for g in range(3):
                e = KEPT4[g][kk]
                recv_s1[(g, kk)].wait_recv()
                work_ref[kk, grows_slice(g), :] = iadd(
                    work_ref[kk, grows_slice(g), :], pack(piece(e, g)))

        s2_recv = {}
        for g in range(3):
            mk = MASKD[GORDER[g][1]]
            for i, e in enumerate(SENT2[g]):
                i_dst = KEPT2[g].index(e ^ mk)
                rr = pltpu.make_async_remote_copy(
                    src_ref=work_ref.at[KEPT4[g].index(e), grows_slice(g), :],
                    dst_ref=r2_ref.at[i_dst, grows_slice(g), :],
                    send_sem=s2s_sems.at[g, i],
                    recv_sem=s2r_sems.at[g, i_dst],
                    device_id=(my ^ mk,),
                    device_id_type=pl.DeviceIdType.MESH,
                )
                rr.start()
                s2_recv[(g, i_dst)] = rr
                all_sends.append(rr)
        for i in range(2):
            for g in range(3):
                e = KEPT2[g][i]
                kk = KEPT4[g].index(e)
                s2_recv[(g, i)].wait_recv()
                work_ref[kk, grows_slice(g), :] = iadd(
                    work_ref[kk, grows_slice(g), :],
                    r2_ref[i, grows_slice(g), :])

        s3_recv = {}
        for g in range(3):
            mk = MASKD[GORDER[g][2]]
            e = SENT1[g][0]
            rr = pltpu.make_async_remote_copy(
                src_ref=work_ref.at[KEPT4[g].index(e), grows_slice(g), :],
                dst_ref=r3_ref.at[grows_slice(g), :],
                send_sem=s3s_sems.at[g],
                recv_sem=s3r_sems.at[g],
                device_id=(my ^ mk,),
                device_id_type=pl.DeviceIdType.MESH,
            )
            rr.start()
            s3_recv[g] = rr
            all_sends.append(rr)
        fc = [None] * 3
        for g in range(3):
            kk = KEPT4[g].index(0)
            s3_recv[g].wait_recv()
            fc[g] = iadd(work_ref[kk, grows_slice(g), :],
                         r3_ref[grows_slice(g), :])

        imax = jnp.max(jnp.stack([
            jnp.maximum(jnp.max(jnp.abs(lanes(fc[g])[0])),
                        jnp.max(jnp.abs(lanes(fc[g])[1])))
            for g in range(3)]))
        local_amax = imax.astype(jnp.float32) * (1.0 / S)
        amax_ref[pl.ds(my, 1)] = jnp.full((1, 8, 128), local_amax,
                                          dtype=jnp.float32)
        ax_rdmas = []
        for off in range(1, N_DEV):
            r = pltpu.make_async_remote_copy(
                src_ref=amax_ref.at[my],
                dst_ref=amax_ref.at[my],
                send_sem=ax_send_sems.at[off],
                recv_sem=ax_recv_sems.at[my],
                device_id=((my + off) % N_DEV,),
                device_id_type=pl.DeviceIdType.MESH,
            )
            r.start()
            ax_rdmas.append(r)
        for off in range(1, N_DEV):
            src = (my + off) % N_DEV
            pltpu.make_async_remote_copy(
                src_ref=amax_ref.at[src],
                dst_ref=amax_ref.at[src],
                send_sem=ax_send_sems.at[off],
                recv_sem=ax_recv_sems.at[src],
                device_id=(my,),
                device_id_type=pl.DeviceIdType.MESH,
            ).wait_recv()
        gmax = jnp.max(amax_ref[:, :, :])

        scale = gmax / 448.0
        inv_scale_s = 448.0 / gmax * (1.0 / S)
        for g in range(3):
            lo, hi = lanes(fc[g])
            for half, c in ((0, lo), (1, hi)):
                yq = jnp.clip(c.astype(jnp.float32) * inv_scale_s,
                              -448.0, 448.0).astype(jnp.float8_e4m3fn)
                out_ref[grows_slice(g), half * n2:(half + 1) * n2] = (
                    yq.astype(jnp.float32) * scale).astype(jnp.bfloat16)

        for r in sb_inflight:
            if r is not None:
                r.wait_send()
        for r in ax_rdmas:
            r.wait_send()
        for r in all_sends:
            r.wait_send()

    return pl.pallas_call(
        body,
        out_shape=jax.ShapeDtypeStruct((m_per, n), jnp.bfloat16),
        in_specs=[
            pl.BlockSpec(memory_space=pl.ANY),
            pl.BlockSpec(memory_space=pl.ANY),
        ],
        out_specs=pl.BlockSpec(memory_space=pltpu.VMEM),
        scratch_shapes=[
            pltpu.VMEM((m, k), jnp.bfloat16),
            pltpu.VMEM((k, n), jnp.bfloat16),
            pltpu.VMEM((2, m_per, k), jnp.float32),
            pltpu.VMEM((4, 176, n2), jnp.int32),
            pltpu.VMEM((4, m_per, n2), jnp.int32),
            pltpu.VMEM((2, m_per, n2), jnp.int32),
            pltpu.VMEM((m_per, n2), jnp.int32),
            pltpu.VMEM((N_DEV, 8, 128), jnp.float32),
            pltpu.SemaphoreType.DMA((2,)),
            pltpu.SemaphoreType.DMA((3, 4)),
            pltpu.SemaphoreType.DMA((3, 4)),
            pltpu.SemaphoreType.DMA((3, 2)),
            pltpu.SemaphoreType.DMA((3, 2)),
            pltpu.SemaphoreType.DMA((3,)),
            pltpu.SemaphoreType.DMA((3,)),
            pltpu.SemaphoreType.DMA((N_DEV,)),
            pltpu.SemaphoreType.DMA((N_DEV,)),
        ],
        compiler_params=pltpu.CompilerParams(collective_id=0),
    )(x, w_mat)


# device time: 85919 ns/iter; 1.1557x vs baseline; 1.0721x over previous
import jax
import jax.numpy as jnp
from jax import lax
from jax.experimental import pallas as pl
from jax.experimental.pallas import tpu as pltpu

N_DEV = 8

GROWS = ((0, 176), (176, 344), (344, 512))
GORDER = ((0, 1, 2), (1, 2, 0), (2, 0, 1))
MASKD = (1, 3, 4)

E_ORDER = (1, 2, 3, 4, 5, 6, 7, 0)


def _D(d, e):
    if d == 0:
        return (e ^ (e >> 1)) & 1
    if d == 1:
        return (e >> 1) & 1
    return (e >> 2) & 1


KEPT4 = tuple(tuple(e for e in range(8) if _D(GORDER[g][0], e) == 0)
              for g in range(3))
KEPT2 = tuple(tuple(e for e in KEPT4[g] if _D(GORDER[g][1], e) == 0)
              for g in range(3))
SENT2 = tuple(tuple(e for e in KEPT4[g] if _D(GORDER[g][1], e) == 1)
              for g in range(3))
SENT1 = tuple(tuple(e for e in KEPT2[g] if _D(GORDER[g][2], e) == 1)
              for g in range(3))

BOUND = 8.0
S = 32760.0 / BOUND


def kernel(x, w_mat):
    m, k = x.shape
    _, n = w_mat.shape
    m_per = m // N_DEV
    n2 = n // 2

    def body(x_ref, w_ref, out_ref, xb_ref, wb_ref, stage_ref, sendbuf_ref,
             work_ref, r2_ref, r3_ref, amax_ref,
             stage_sems, s1s_sems, s1r_sems, s2s_sems, s2r_sems,
             s3s_sems, s3r_sems, ax_send_sems, ax_recv_sems):
        my = lax.axis_index("i")

        def w_copy(qt):
            cp = pltpu.make_async_copy(
                w_ref.at[:, pl.ds(qt * 512, 512)],
                stage_ref.at[qt % 2], stage_sems.at[qt % 2])
            cp.start()
            return cp

        def x_copy(t):
            c = my ^ E_ORDER[t]
            cp = pltpu.make_async_copy(
                x_ref.at[pl.ds(c * m_per, m_per), :],
                stage_ref.at[t % 2], stage_sems.at[t % 2])
            cp.start()
            return cp

        wcp = [None] * 4
        wcp[0] = w_copy(0)
        wcp[1] = w_copy(1)
        xcp = [None] * N_DEV

        def x_cast(t):
            xcp[t].wait()
            c = my ^ E_ORDER[t]
            xb_ref[pl.ds(c * m_per, m_per), :] = (
                stage_ref[t % 2].astype(jnp.bfloat16))
            if t + 2 < N_DEV:
                xcp[t + 2] = x_copy(t + 2)

        barrier_sem = pltpu.get_barrier_semaphore()
        for mk in MASKD:
            pl.semaphore_signal(
                barrier_sem, inc=1,
                device_id=(my ^ mk,), device_id_type=pl.DeviceIdType.MESH,
            )

        for qt in range(4):
            wcp[qt].wait()
            wb_ref[:, qt * 512:(qt + 1) * 512] = (
                stage_ref[qt % 2].astype(jnp.bfloat16))
            if qt + 2 < 4:
                wcp[qt + 2] = w_copy(qt + 2)
            else:
                xcp[qt - 2] = x_copy(qt - 2)

        pl.semaphore_wait(barrier_sem, 3)

        def grows_slice(g):
            g0, g1 = GROWS[g]
            return pl.ds(g0, g1 - g0)

        def piece(e, g):
            g0, g1 = GROWS[g]
            j = my ^ e
            return jnp.dot(
                xb_ref[pl.ds(j * m_per + g0, g1 - g0), :], wb_ref[:, :],
                preferred_element_type=jnp.float32)

        def enc(v):
            u = jnp.clip(v * S, -32760.0, 32760.0) + 32768.5
            return u.astype(jnp.int32) - 32768

        def pack(v):
            return (jnp.left_shift(enc(v[:, n2:]), 16)
                    | (enc(v[:, :n2]) & 0xFFFF))

        def lanes(pp):
            lo = jnp.right_shift(jnp.left_shift(pp, 16), 16)
            hi = jnp.right_shift(pp, 16)
            return lo, hi

        def iadd(a, b):
            alo, ahi = lanes(a)
            blo, bhi = lanes(b)
            return (jnp.left_shift(ahi + bhi, 16) | ((alo + blo) & 0xFFFF))

        all_sends = []

        recv_s1 = {}
        for g in range(3):
            for kk in range(4):
                recv_s1[(g, kk)] = pltpu.make_async_remote_copy(
                    src_ref=work_ref.at[kk, grows_slice(g), :],
                    dst_ref=work_ref.at[kk, grows_slice(g), :],
                    send_sem=s1s_sems.at[g, kk],
                    recv_sem=s1r_sems.at[g, kk],
                    device_id=(my ^ MASKD[GORDER[g][0]],),
                    device_id_type=pl.DeviceIdType.MESH,
                )
        sb_inflight = [None] * 6
        sb_i = 0
        scount = [0, 0, 0]
        for t in range(N_DEV):
            x_cast(t)
            e = E_ORDER[t]
            if e == 0:
                continue
            for g in range(3):
                d0 = GORDER[g][0]
                if _D(d0, e) != 1:
                    continue
                mk = MASKD[d0]
                k_dst = KEPT4[g].index(e ^ mk)
                g0, g1 = GROWS[g]
                rows = g1 - g0
                if sb_inflight[sb_i] is not None:
                    sb_inflight[sb_i].wait_send()
                sendbuf_ref[sb_i, 0:rows, :] = pack(piece(e, g))
                rr = pltpu.make_async_remote_copy(
                    src_ref=sendbuf_ref.at[sb_i, pl.ds(0, rows), :],
                    dst_ref=work_ref.at[k_dst, grows_slice(g), :],
                    send_sem=s1s_sems.at[g, scount[g]],
                    recv_sem=s1r_sems.at[g, k_dst],
                    device_id=(my ^ mk,),
                    device_id_type=pl.DeviceIdType.MESH,
                )
                rr.start()
                sb_inflight[sb_i] = rr
                sb_i = (sb_i + 1) % 6
                scount[g] += 1
        for kk in range(4):
            for g in range(3):
                e = KEPT4[g][kk]
                recv_s1[(g, kk)].wait_recv()
                work_ref[kk, grows_slice(g), :] = iadd(
                    work_ref[kk, grows_slice(g), :], pack(piece(e, g)))

        s2_recv = {}
        for g in range(3):
            mk = MASKD[GORDER[g][1]]
            for i, e in enumerate(SENT2[g]):
                i_dst = KEPT2[g].index(e ^ mk)
                rr = pltpu.make_async_remote_copy(
                    src_ref=work_ref.at[KEPT4[g].index(e), grows_slice(g), :],
                    dst_ref=r2_ref.at[i_dst, grows_slice(g), :],
                    send_sem=s2s_sems.at[g, i],
                    recv_sem=s2r_sems.at[g, i_dst],
                    device_id=(my ^ mk,),
                    device_id_type=pl.DeviceIdType.MESH,
                )
                rr.start()
                s2_recv[(g, i_dst)] = rr
                all_sends.append(rr)
        for i in range(2):
            for g in range(3):
                e = KEPT2[g][i]
                kk = KEPT4[g].index(e)
                s2_recv[(g, i)].wait_recv()
                work_ref[kk, grows_slice(g), :] = iadd(
                    work_ref[kk, grows_slice(g), :],
                    r2_ref[i, grows_slice(g), :])

        s3_recv = {}
        for g in range(3):
            mk = MASKD[GORDER[g][2]]
            e = SENT1[g][0]
            rr = pltpu.make_async_remote_copy(
                src_ref=work_ref.at[KEPT4[g].index(e), grows_slice(g), :],
                dst_ref=r3_ref.at[grows_slice(g), :],
                send_sem=s3s_sems.at[g],
                recv_sem=s3r_sems.at[g],
                device_id=(my ^ mk,),
                device_id_type=pl.DeviceIdType.MESH,
            )
            rr.start()
            s3_recv[g] = rr
            all_sends.append(rr)
        fc = [None] * 3
        for g in range(3):
            kk = KEPT4[g].index(0)
            s3_recv[g].wait_recv()
            fc[g] = iadd(work_ref[kk, grows_slice(g), :],
                         r3_ref[grows_slice(g), :])

        imax = jnp.max(jnp.stack([
            jnp.maximum(jnp.max(jnp.abs(lanes(fc[g])[0])),
                        jnp.max(jnp.abs(lanes(fc[g])[1])))
            for g in range(3)]))
        local_amax = imax.astype(jnp.float32) * (1.0 / S)
        amax_ref[pl.ds(my, 1)] = jnp.full((1, 8, 128), local_amax,
                                          dtype=jnp.float32)
        ax_rdmas = []
        for off in range(1, N_DEV):
            r = pltpu.make_async_remote_copy(
                src_ref=amax_ref.at[my],
                dst_ref=amax_ref.at[my],
                send_sem=ax_send_sems.at[off],
                recv_sem=ax_recv_sems.at[my],
                device_id=((my + off) % N_DEV,),
                device_id_type=pl.DeviceIdType.MESH,
            )
            r.start()
            ax_rdmas.append(r)
        for off in range(1, N_DEV):
            src = (my + off) % N_DEV
            pltpu.make_async_remote_copy(
                src_ref=amax_ref.at[src],
                dst_ref=amax_ref.at[src],
                send_sem=ax_send_sems.at[off],
                recv_sem=ax_recv_sems.at[src],
                device_id=(my,),
                device_id_type=pl.DeviceIdType.MESH,
            ).wait_recv()
        gmax = jnp.max(amax_ref[:, :, :])

        scale = gmax / 448.0
        inv_scale_s = 448.0 / gmax * (1.0 / S)
        for g in range(3):
            lo, hi = lanes(fc[g])
            for half, c in ((0, lo), (1, hi)):
                yq = jnp.clip(c.astype(jnp.float32) * inv_scale_s,
                              -448.0, 448.0).astype(jnp.float8_e4m3fn)
                out_ref[grows_slice(g), half * n2:(half + 1) * n2] = (
                    yq.astype(jnp.float32) * scale).astype(jnp.bfloat16)

        for r in sb_inflight:
            if r is not None:
                r.wait_send()
        for r in ax_rdmas:
            r.wait_send()
        for r in all_sends:
            r.wait_send()

    return pl.pallas_call(
        body,
        out_shape=jax.ShapeDtypeStruct((m_per, n), jnp.bfloat16),
        in_specs=[
            pl.BlockSpec(memory_space=pl.ANY),
            pl.BlockSpec(memory_space=pl.ANY),
        ],
        out_specs=pl.BlockSpec(memory_space=pltpu.VMEM),
        scratch_shapes=[
            pltpu.VMEM((m, k), jnp.bfloat16),
            pltpu.VMEM((k, n), jnp.bfloat16),
            pltpu.VMEM((2, m_per, k), jnp.float32),
            pltpu.VMEM((6, 176, n2), jnp.int32),
            pltpu.VMEM((4, m_per, n2), jnp.int32),
            pltpu.VMEM((2, m_per, n2), jnp.int32),
            pltpu.VMEM((m_per, n2), jnp.int32),
            pltpu.VMEM((N_DEV, 8, 128), jnp.float32),
            pltpu.SemaphoreType.DMA((2,)),
            pltpu.SemaphoreType.DMA((3, 4)),
            pltpu.SemaphoreType.DMA((3, 4)),
            pltpu.SemaphoreType.DMA((3, 2)),
            pltpu.SemaphoreType.DMA((3, 2)),
            pltpu.SemaphoreType.DMA((3,)),
            pltpu.SemaphoreType.DMA((3,)),
            pltpu.SemaphoreType.DMA((N_DEV,)),
            pltpu.SemaphoreType.DMA((N_DEV,)),
        ],
        compiler_params=pltpu.CompilerParams(collective_id=0),
    )(x, w_mat)
